# Initial kernel scaffold; baseline (speedup 1.0000x reference)
import jax
import jax.numpy as jnp
from jax.experimental import pallas as pl


def kernel(node_input, node_attr, node_deg, edge_src, edge_dst, edge_attr, edge_length_embedded, numb, n, W_in, W_mask, W1, W2, W_out):
    raise NotImplementedError("write your pallas kernel here")


# speedup vs baseline: 2.2305x; 2.2305x over previous
import functools
import math

import jax
import jax.numpy as jnp
from jax import lax
from jax.experimental import pallas as pl
from jax.experimental.pallas import tpu as pltpu
from jax.experimental.pallas import tpu_sc as plsc

N = 10000
E = 320000
D_IN = 128
D_ATTR = 4
D_OUT = 128
N_BASIS = 10
RADIAL = 100

NORM_IN = 1.0 / math.sqrt(D_IN * D_ATTR)
C_S = math.sin(math.pi / 8.0)
C_X = math.cos(math.pi / 8.0)

BN = 400


def _node_feat_body(ni_ref, attr_ref, deg_ref, w_ref, out_ref):
    acc = jnp.zeros((BN, D_IN), jnp.float32)
    for b in range(D_ATTR):
        acc += jnp.dot(ni_ref[:] * attr_ref[:, b:b + 1], w_ref[b],
                       preferred_element_type=jnp.float32)
    out_ref[:] = acc * NORM_IN / jnp.sqrt(deg_ref[:])


def _node_features(node_input, node_attr, node_deg, W_in_t):
    return pl.pallas_call(
        _node_feat_body,
        grid=(N // BN,),
        in_specs=[
            pl.BlockSpec((BN, D_IN), lambda i: (i, 0)),
            pl.BlockSpec((BN, D_ATTR), lambda i: (i, 0)),
            pl.BlockSpec((BN, 1), lambda i: (i, 0)),
            pl.BlockSpec((D_ATTR, D_IN, D_IN), lambda i: (0, 0, 0)),
        ],
        out_specs=pl.BlockSpec((BN, D_IN), lambda i: (i, 0)),
        out_shape=jax.ShapeDtypeStruct((N, D_IN), jnp.float32),
    )(node_input, node_attr, node_deg, W_in_t)


BE = 3200


def _edge_w_body(elem_ref, ea_ref, w1_ref, w2_ref, out_ref):
    h = jax.nn.silu(jnp.dot(elem_ref[:], w1_ref[:],
                            preferred_element_type=jnp.float32)
                    * (1.0 / math.sqrt(N_BASIS)))
    ew = jnp.dot(h, w2_ref[:], preferred_element_type=jnp.float32)
    out_ref[:] = ew * (1.0 / math.sqrt(RADIAL)) * ea_ref[:]


def _edge_weights(edge_length_embedded, edge_attr, W1, W2):
    return pl.pallas_call(
        _edge_w_body,
        grid=(E // BE,),
        in_specs=[
            pl.BlockSpec((BE, N_BASIS), lambda i: (i, 0)),
            pl.BlockSpec((BE, 1), lambda i: (i, 0)),
            pl.BlockSpec((N_BASIS, RADIAL), lambda i: (0, 0)),
            pl.BlockSpec((RADIAL, D_IN), lambda i: (0, 0)),
        ],
        out_specs=pl.BlockSpec((BE, D_IN), lambda i: (i, 0)),
        out_shape=jax.ShapeDtypeStruct((E, D_IN), jnp.float32),
    )(edge_length_embedded, edge_attr, W1, W2)


NUM_CORES = 2
NUM_SUBCORES = 16
NUM_TILES = NUM_CORES * NUM_SUBCORES
EDGES_PER_TILE = E // NUM_TILES
CHUNK = 80
NCHUNK = EDGES_PER_TILE // CHUNK
ROWS_PER_TILE = N // NUM_SUBCORES


def _sc_edge_kernel(nf_hbm, w_hbm, src_hbm, dst_hbm, zeros_hbm, out_hbm,
                    src_v, dst_v, g_v, w_v, agg_sh, sem):
    c = lax.axis_index("c")
    s = lax.axis_index("s")
    wid = c * NUM_SUBCORES + s
    pltpu.sync_copy(zeros_hbm, agg_sh.at[pl.ds(s * ROWS_PER_TILE, ROWS_PER_TILE)])
    plsc.subcore_barrier()
    base_e = wid * EDGES_PER_TILE

    def body(i, carry):
        off = base_e + i * CHUNK
        pltpu.sync_copy(src_hbm.at[pl.ds(off, CHUNK)], src_v)
        pltpu.sync_copy(dst_hbm.at[pl.ds(off, CHUNK)], dst_v)
        pltpu.async_copy(nf_hbm.at[src_v], g_v, sem).wait()
        pltpu.sync_copy(w_hbm.at[pl.ds(off, CHUNK)], w_v)

        def rowfn(r, carry2):
            for j in range(D_IN // 16):
                sl = pl.ds(j * 16, 16)
                w_v[r, sl] = w_v[r, sl] * g_v[r, sl]
            return carry2

        lax.fori_loop(0, CHUNK, rowfn, 0)
        pltpu.sync_copy(w_v, agg_sh.at[dst_v], add=True)
        return carry

    lax.fori_loop(0, NCHUNK, body, 0)
    plsc.subcore_barrier()

    @pl.when(s == 0)
    def _():
        pltpu.sync_copy(agg_sh, out_hbm.at[c])


def _sc_aggregate(nf, w_edge, edge_src, edge_dst):
    mesh = plsc.VectorSubcoreMesh(core_axis_name="c", subcore_axis_name="s",
                                  num_cores=NUM_CORES,
                                  num_subcores=NUM_SUBCORES)
    zeros = jnp.zeros((ROWS_PER_TILE, D_IN), jnp.float32)
    call = functools.partial(
        pl.kernel,
        out_type=jax.ShapeDtypeStruct((NUM_CORES, N, D_IN), jnp.float32),
        mesh=mesh,
        scratch_types=[
            pltpu.VMEM((CHUNK,), jnp.int32),
            pltpu.VMEM((CHUNK,), jnp.int32),
            pltpu.VMEM((CHUNK, D_IN), jnp.float32),
            pltpu.VMEM((CHUNK, D_IN), jnp.float32),
            pltpu.VMEM_SHARED((N, D_IN), jnp.float32),
            pltpu.SemaphoreType.DMA,
        ],
    )(_sc_edge_kernel)
    return call(nf, w_edge, edge_src, edge_dst, zeros)


def _output_body(agg_ref, ni_ref, attr_ref, deg_ref, wm_ref, wo_ref, out_ref):
    a = (agg_ref[0] + agg_ref[1]) / jnp.sqrt(deg_ref[:])
    accm = jnp.zeros((BN, D_OUT), jnp.float32)
    acco = jnp.zeros((BN, D_OUT), jnp.float32)
    for b in range(D_ATTR):
        ab = attr_ref[:, b:b + 1]
        accm += jnp.dot(ni_ref[:] * ab, wm_ref[b],
                        preferred_element_type=jnp.float32)
        acco += jnp.dot(a * ab, wo_ref[b],
                        preferred_element_type=jnp.float32)
    out_ref[:] = (C_S * NORM_IN) * accm + (C_X * NORM_IN) * acco


def _output(agg2, node_input, node_attr, node_deg, W_mask_t, W_out_t):
    return pl.pallas_call(
        _output_body,
        grid=(N // BN,),
        in_specs=[
            pl.BlockSpec((NUM_CORES, BN, D_IN), lambda i: (0, i, 0)),
            pl.BlockSpec((BN, D_IN), lambda i: (i, 0)),
            pl.BlockSpec((BN, D_ATTR), lambda i: (i, 0)),
            pl.BlockSpec((BN, 1), lambda i: (i, 0)),
            pl.BlockSpec((D_ATTR, D_IN, D_OUT), lambda i: (0, 0, 0)),
            pl.BlockSpec((D_ATTR, D_IN, D_OUT), lambda i: (0, 0, 0)),
        ],
        out_specs=pl.BlockSpec((BN, D_OUT), lambda i: (i, 0)),
        out_shape=jax.ShapeDtypeStruct((N, D_OUT), jnp.float32),
    )(agg2, node_input, node_attr, node_deg, W_mask_t, W_out_t)


def kernel(node_input, node_attr, node_deg, edge_src, edge_dst, edge_attr,
           edge_length_embedded, numb, n, W_in, W_mask, W1, W2, W_out):
    edge_src = edge_src.astype(jnp.int32)
    edge_dst = edge_dst.astype(jnp.int32)
    W_in_t = jnp.transpose(W_in, (1, 0, 2))
    W_mask_t = jnp.transpose(W_mask, (1, 0, 2))
    W_out_t = jnp.transpose(W_out, (1, 0, 2))

    nf = _node_features(node_input, node_attr, node_deg, W_in_t)
    w_edge = _edge_weights(edge_length_embedded, edge_attr, W1, W2)
    agg2 = _sc_aggregate(nf, w_edge, edge_src, edge_dst)
    return _output(agg2, node_input, node_attr, node_deg, W_mask_t, W_out_t)


# speedup vs baseline: 3.1301x; 1.4033x over previous
import functools
import math

import jax
import jax.numpy as jnp
from jax import lax
from jax.experimental import pallas as pl
from jax.experimental.pallas import tpu as pltpu
from jax.experimental.pallas import tpu_sc as plsc

N = 10000
E = 320000
D_IN = 128
D_ATTR = 4
D_OUT = 128
N_BASIS = 10
RADIAL = 100

NORM_IN = 1.0 / math.sqrt(D_IN * D_ATTR)
C_S = math.sin(math.pi / 8.0)
C_X = math.cos(math.pi / 8.0)

BN = 400


def _node_feat_body(ni_ref, attr_ref, deg_ref, w_ref, out_ref):
    acc = jnp.zeros((BN, D_IN), jnp.float32)
    for b in range(D_ATTR):
        acc += jnp.dot(ni_ref[:] * attr_ref[:, b:b + 1], w_ref[b],
                       preferred_element_type=jnp.float32)
    out_ref[:] = acc * NORM_IN / jnp.sqrt(deg_ref[:])


def _node_features(node_input, node_attr, node_deg, W_in_t):
    return pl.pallas_call(
        _node_feat_body,
        grid=(N // BN,),
        in_specs=[
            pl.BlockSpec((BN, D_IN), lambda i: (i, 0)),
            pl.BlockSpec((BN, D_ATTR), lambda i: (i, 0)),
            pl.BlockSpec((BN, 1), lambda i: (i, 0)),
            pl.BlockSpec((D_ATTR, D_IN, D_IN), lambda i: (0, 0, 0)),
        ],
        out_specs=pl.BlockSpec((BN, D_IN), lambda i: (i, 0)),
        out_shape=jax.ShapeDtypeStruct((N, D_IN), jnp.float32),
    )(node_input, node_attr, node_deg, W_in_t)


BE = 3200


def _edge_w_body(elem_ref, ea_ref, w1_ref, w2_ref, out_ref):
    h = jax.nn.silu(jnp.dot(elem_ref[:], w1_ref[:],
                            preferred_element_type=jnp.float32)
                    * (1.0 / math.sqrt(N_BASIS)))
    ew = jnp.dot(h, w2_ref[:], preferred_element_type=jnp.float32)
    out_ref[:] = ew * (1.0 / math.sqrt(RADIAL)) * ea_ref[:]


def _edge_weights(edge_length_embedded, edge_attr, W1, W2):
    return pl.pallas_call(
        _edge_w_body,
        grid=(E // BE,),
        in_specs=[
            pl.BlockSpec((BE, N_BASIS), lambda i: (i, 0)),
            pl.BlockSpec((BE, 1), lambda i: (i, 0)),
            pl.BlockSpec((N_BASIS, RADIAL), lambda i: (0, 0)),
            pl.BlockSpec((RADIAL, D_IN), lambda i: (0, 0)),
        ],
        out_specs=pl.BlockSpec((BE, D_IN), lambda i: (i, 0)),
        out_shape=jax.ShapeDtypeStruct((E, D_IN), jnp.float32),
    )(edge_length_embedded, edge_attr, W1, W2)


NUM_CORES = 2
NUM_SUBCORES = 16
NUM_TILES = NUM_CORES * NUM_SUBCORES
EDGES_PER_TILE = E // NUM_TILES
CHUNK = 40
NCHUNK = EDGES_PER_TILE // CHUNK
NSLOT = 3
ROWS_PER_TILE = N // NUM_SUBCORES


def _sc_edge_kernel(nf_hbm, w_hbm, src_hbm, dst_hbm, zeros_hbm, out_hbm,
                    src_v, dst_v, g_v, w_v, agg_sh, *sems):
    sem_idx = sems[0:NSLOT]
    sem_in = sems[NSLOT:2 * NSLOT]
    sem_out = sems[2 * NSLOT:3 * NSLOT]
    c = lax.axis_index("c")
    s = lax.axis_index("s")
    wid = c * NUM_SUBCORES + s
    base_e = wid * EDGES_PER_TILE

    pltpu.sync_copy(zeros_hbm, agg_sh.at[pl.ds(s * ROWS_PER_TILE, ROWS_PER_TILE)])
    plsc.subcore_barrier()

    def when(cond, fn):
        if isinstance(cond, bool):
            if cond:
                fn()
        else:
            pl.when(cond)(fn)

    def issue_idx(i, b):
        pltpu.async_copy(src_hbm.at[pl.ds(base_e + i * CHUNK, CHUNK)],
                         src_v.at[b], sem_idx[b])
        pltpu.async_copy(dst_hbm.at[pl.ds(base_e + i * CHUNK, CHUNK)],
                         dst_v.at[b], sem_idx[b])

    def wait_idx(b):
        pltpu.make_async_copy(src_hbm.at[pl.ds(0, CHUNK)], src_v.at[b],
                              sem_idx[b]).wait()
        pltpu.make_async_copy(src_hbm.at[pl.ds(0, CHUNK)], dst_v.at[b],
                              sem_idx[b]).wait()

    def issue_in(i, b):
        pltpu.async_copy(nf_hbm.at[src_v.at[b]], g_v.at[b], sem_in[b])
        pltpu.async_copy(w_hbm.at[pl.ds(base_e + i * CHUNK, CHUNK)],
                         w_v.at[b], sem_in[b])

    def wait_in(b):
        pltpu.make_async_copy(w_hbm.at[pl.ds(0, CHUNK)], g_v.at[b],
                              sem_in[b]).wait()
        pltpu.make_async_copy(w_hbm.at[pl.ds(0, CHUNK)], w_v.at[b],
                              sem_in[b]).wait()

    def compute(b):
        def rowfn(r, carry2):
            for j in range(D_IN // 16):
                sl = pl.ds(j * 16, 16)
                w_v[b, r, sl] = w_v[b, r, sl] * g_v[b, r, sl]
            return carry2

        lax.fori_loop(0, CHUNK, rowfn, 0)

    def scatter(b):
        pltpu.async_copy(w_v.at[b], agg_sh.at[dst_v.at[b]], sem_out[b],
                         add=True)

    def wait_out(b):
        pltpu.make_async_copy(w_hbm.at[pl.ds(0, CHUNK)], w_v.at[b],
                              sem_out[b]).wait()

    def step(i, u):
        def prefetch_data():
            wait_idx((u + 1) % NSLOT)
            issue_in(i + 1, (u + 1) % NSLOT)

        when(i + 1 < NCHUNK, prefetch_data)
        wait_in(u)
        compute(u)
        scatter(u)
        when(i >= 1, lambda: wait_out((u + NSLOT - 1) % NSLOT))
        when(i + 2 < NCHUNK, lambda: issue_idx(i + 2, (u + 2) % NSLOT))

    issue_idx(0, 0)
    issue_idx(1, 1)
    wait_idx(0)
    issue_in(0, 0)

    n_main = (NCHUNK - 1) // NSLOT

    def body(k, carry):
        for u in range(NSLOT):
            step(k * NSLOT + u, u)
        return carry

    lax.fori_loop(0, n_main, body, 0)
    for i in range(n_main * NSLOT, NCHUNK):
        step(i, i % NSLOT)
    wait_out((NCHUNK - 1) % NSLOT)

    plsc.subcore_barrier()

    @pl.when(s == 0)
    def _():
        pltpu.sync_copy(agg_sh, out_hbm.at[c])


def _sc_aggregate(nf, w_edge, edge_src, edge_dst):
    mesh = plsc.VectorSubcoreMesh(core_axis_name="c", subcore_axis_name="s",
                                  num_cores=NUM_CORES,
                                  num_subcores=NUM_SUBCORES)
    zeros = jnp.zeros((ROWS_PER_TILE, D_IN), jnp.float32)
    call = functools.partial(
        pl.kernel,
        out_type=jax.ShapeDtypeStruct((NUM_CORES, N, D_IN), jnp.float32),
        mesh=mesh,
        scratch_types=(
            [
                pltpu.VMEM((NSLOT, CHUNK), jnp.int32),
                pltpu.VMEM((NSLOT, CHUNK), jnp.int32),
                pltpu.VMEM((NSLOT, CHUNK, D_IN), jnp.float32),
                pltpu.VMEM((NSLOT, CHUNK, D_IN), jnp.float32),
                pltpu.VMEM_SHARED((N, D_IN), jnp.float32),
            ]
            + [pltpu.SemaphoreType.DMA] * (3 * NSLOT)
        ),
    )(_sc_edge_kernel)
    return call(nf, w_edge, edge_src, edge_dst, zeros)


def _output_body(agg_ref, ni_ref, attr_ref, deg_ref, wm_ref, wo_ref, out_ref):
    a = (agg_ref[0] + agg_ref[1]) / jnp.sqrt(deg_ref[:])
    accm = jnp.zeros((BN, D_OUT), jnp.float32)
    acco = jnp.zeros((BN, D_OUT), jnp.float32)
    for b in range(D_ATTR):
        ab = attr_ref[:, b:b + 1]
        accm += jnp.dot(ni_ref[:] * ab, wm_ref[b],
                        preferred_element_type=jnp.float32)
        acco += jnp.dot(a * ab, wo_ref[b],
                        preferred_element_type=jnp.float32)
    out_ref[:] = (C_S * NORM_IN) * accm + (C_X * NORM_IN) * acco


def _output(agg2, node_input, node_attr, node_deg, W_mask_t, W_out_t):
    return pl.pallas_call(
        _output_body,
        grid=(N // BN,),
        in_specs=[
            pl.BlockSpec((2, BN, D_IN), lambda i: (0, i, 0)),
            pl.BlockSpec((BN, D_IN), lambda i: (i, 0)),
            pl.BlockSpec((BN, D_ATTR), lambda i: (i, 0)),
            pl.BlockSpec((BN, 1), lambda i: (i, 0)),
            pl.BlockSpec((D_ATTR, D_IN, D_OUT), lambda i: (0, 0, 0)),
            pl.BlockSpec((D_ATTR, D_IN, D_OUT), lambda i: (0, 0, 0)),
        ],
        out_specs=pl.BlockSpec((BN, D_OUT), lambda i: (i, 0)),
        out_shape=jax.ShapeDtypeStruct((N, D_OUT), jnp.float32),
    )(agg2, node_input, node_attr, node_deg, W_mask_t, W_out_t)


def kernel(node_input, node_attr, node_deg, edge_src, edge_dst, edge_attr,
           edge_length_embedded, numb, n, W_in, W_mask, W1, W2, W_out):
    edge_src = edge_src.astype(jnp.int32)
    edge_dst = edge_dst.astype(jnp.int32)
    W_in_t = jnp.transpose(W_in, (1, 0, 2))
    W_mask_t = jnp.transpose(W_mask, (1, 0, 2))
    W_out_t = jnp.transpose(W_out, (1, 0, 2))

    nf = _node_features(node_input, node_attr, node_deg, W_in_t)
    w_edge = _edge_weights(edge_length_embedded, edge_attr, W1, W2)
    agg2 = _sc_aggregate(nf, w_edge, edge_src, edge_dst)
    return _output(agg2, node_input, node_attr, node_deg, W_mask_t, W_out_t)


# speedup vs baseline: 3.1521x; 1.0070x over previous
import functools
import math

import jax
import jax.numpy as jnp
from jax import lax
from jax.experimental import pallas as pl
from jax.experimental.pallas import tpu as pltpu
from jax.experimental.pallas import tpu_sc as plsc

N = 10000
E = 320000
D_IN = 128
D_ATTR = 4
D_OUT = 128
N_BASIS = 10
RADIAL = 100

NORM_IN = 1.0 / math.sqrt(D_IN * D_ATTR)
C_S = math.sin(math.pi / 8.0)
C_X = math.cos(math.pi / 8.0)

GSTEPS = 125
BN2 = N // GSTEPS
BE2 = E // GSTEPS


def _front_body(ni_ref, attr_ref, deg_ref, win_ref, elem_ref, ea_ref,
                w1_ref, w2_ref, nf_ref, w_ref):
    acc = jnp.zeros((BN2, D_IN), jnp.float32)
    for b in range(D_ATTR):
        acc += jnp.dot(ni_ref[:] * attr_ref[:, b:b + 1], win_ref[b],
                       preferred_element_type=jnp.float32)
    nf_ref[:] = acc * NORM_IN / jnp.sqrt(deg_ref[:])
    h = jax.nn.silu(jnp.dot(elem_ref[:], w1_ref[:],
                            preferred_element_type=jnp.float32)
                    * (1.0 / math.sqrt(N_BASIS)))
    ew = jnp.dot(h, w2_ref[:], preferred_element_type=jnp.float32)
    w_ref[:] = ew * (1.0 / math.sqrt(RADIAL)) * ea_ref[:]


def _front(node_input, node_attr, node_deg, W_in_t,
           edge_length_embedded, edge_attr, W1, W2):
    return pl.pallas_call(
        _front_body,
        grid=(GSTEPS,),
        in_specs=[
            pl.BlockSpec((BN2, D_IN), lambda i: (i, 0)),
            pl.BlockSpec((BN2, D_ATTR), lambda i: (i, 0)),
            pl.BlockSpec((BN2, 1), lambda i: (i, 0)),
            pl.BlockSpec((D_ATTR, D_IN, D_IN), lambda i: (0, 0, 0)),
            pl.BlockSpec((BE2, N_BASIS), lambda i: (i, 0)),
            pl.BlockSpec((BE2, 1), lambda i: (i, 0)),
            pl.BlockSpec((N_BASIS, RADIAL), lambda i: (0, 0)),
            pl.BlockSpec((RADIAL, D_IN), lambda i: (0, 0)),
        ],
        out_specs=[
            pl.BlockSpec((BN2, D_IN), lambda i: (i, 0)),
            pl.BlockSpec((BE2, D_IN), lambda i: (i, 0)),
        ],
        out_shape=[
            jax.ShapeDtypeStruct((N, D_IN), jnp.float32),
            jax.ShapeDtypeStruct((E, D_IN), jnp.float32),
        ],
    )(node_input, node_attr, node_deg, W_in_t,
      edge_length_embedded, edge_attr, W1, W2)


NUM_CORES = 2
NUM_SUBCORES = 16
NUM_TILES = NUM_CORES * NUM_SUBCORES
EDGES_PER_TILE = E // NUM_TILES
CHUNK = 40
NCHUNK = EDGES_PER_TILE // CHUNK
NSLOT = 3
ROWS_PER_TILE = N // NUM_SUBCORES


def _sc_edge_kernel(nf_hbm, w_hbm, src_hbm, dst_hbm, zeros_hbm, out_hbm,
                    src_v, dst_v, g_v, w_v, agg_sh, *sems):
    sem_idx = sems[0:NSLOT]
    sem_in = sems[NSLOT:2 * NSLOT]
    sem_out = sems[2 * NSLOT:3 * NSLOT]
    c = lax.axis_index("c")
    s = lax.axis_index("s")
    wid = c * NUM_SUBCORES + s
    base_e = wid * EDGES_PER_TILE

    pltpu.sync_copy(zeros_hbm, agg_sh.at[pl.ds(s * ROWS_PER_TILE, ROWS_PER_TILE)])
    plsc.subcore_barrier()

    def when(cond, fn):
        if isinstance(cond, bool):
            if cond:
                fn()
        else:
            pl.when(cond)(fn)

    def issue_idx(i, b):
        pltpu.async_copy(src_hbm.at[pl.ds(base_e + i * CHUNK, CHUNK)],
                         src_v.at[b], sem_idx[b])
        pltpu.async_copy(dst_hbm.at[pl.ds(base_e + i * CHUNK, CHUNK)],
                         dst_v.at[b], sem_idx[b])

    def wait_idx(b):
        pltpu.make_async_copy(src_hbm.at[pl.ds(0, CHUNK)], src_v.at[b],
                              sem_idx[b]).wait()
        pltpu.make_async_copy(src_hbm.at[pl.ds(0, CHUNK)], dst_v.at[b],
                              sem_idx[b]).wait()

    def issue_in(i, b):
        pltpu.async_copy(nf_hbm.at[src_v.at[b]], g_v.at[b], sem_in[b])
        pltpu.async_copy(w_hbm.at[pl.ds(base_e + i * CHUNK, CHUNK)],
                         w_v.at[b], sem_in[b])

    def wait_in(b):
        pltpu.make_async_copy(w_hbm.at[pl.ds(0, CHUNK)], g_v.at[b],
                              sem_in[b]).wait()
        pltpu.make_async_copy(w_hbm.at[pl.ds(0, CHUNK)], w_v.at[b],
                              sem_in[b]).wait()

    def compute(b):
        def rowfn(r, carry2):
            for j in range(D_IN // 16):
                sl = pl.ds(j * 16, 16)
                w_v[b, r, sl] = w_v[b, r, sl] * g_v[b, r, sl]
            return carry2

        lax.fori_loop(0, CHUNK, rowfn, 0)

    def scatter(b):
        pltpu.async_copy(w_v.at[b], agg_sh.at[dst_v.at[b]], sem_out[b],
                         add=True)

    def wait_out(b):
        pltpu.make_async_copy(w_hbm.at[pl.ds(0, CHUNK)], w_v.at[b],
                              sem_out[b]).wait()

    def step(i, u):
        def prefetch_data():
            wait_idx((u + 1) % NSLOT)
            issue_in(i + 1, (u + 1) % NSLOT)

        when(i + 1 < NCHUNK, prefetch_data)
        wait_in(u)
        compute(u)
        scatter(u)
        when(i >= 1, lambda: wait_out((u + NSLOT - 1) % NSLOT))
        when(i + 2 < NCHUNK, lambda: issue_idx(i + 2, (u + 2) % NSLOT))

    issue_idx(0, 0)
    issue_idx(1, 1)
    wait_idx(0)
    issue_in(0, 0)

    n_main = (NCHUNK - 1) // NSLOT

    def body(k, carry):
        for u in range(NSLOT):
            step(k * NSLOT + u, u)
        return carry

    lax.fori_loop(0, n_main, body, 0)
    for i in range(n_main * NSLOT, NCHUNK):
        step(i, i % NSLOT)
    wait_out((NCHUNK - 1) % NSLOT)

    plsc.subcore_barrier()

    @pl.when(s == 0)
    def _():
        pltpu.sync_copy(agg_sh, out_hbm.at[c])


def _sc_aggregate(nf, w_edge, edge_src, edge_dst):
    mesh = plsc.VectorSubcoreMesh(core_axis_name="c", subcore_axis_name="s",
                                  num_cores=NUM_CORES,
                                  num_subcores=NUM_SUBCORES)
    zeros = jnp.zeros((ROWS_PER_TILE, D_IN), jnp.float32)
    call = functools.partial(
        pl.kernel,
        out_type=jax.ShapeDtypeStruct((NUM_CORES, N, D_IN), jnp.float32),
        mesh=mesh,
        scratch_types=(
            [
                pltpu.VMEM((NSLOT, CHUNK), jnp.int32),
                pltpu.VMEM((NSLOT, CHUNK), jnp.int32),
                pltpu.VMEM((NSLOT, CHUNK, D_IN), jnp.float32),
                pltpu.VMEM((NSLOT, CHUNK, D_IN), jnp.float32),
                pltpu.VMEM_SHARED((N, D_IN), jnp.float32),
            ]
            + [pltpu.SemaphoreType.DMA] * (3 * NSLOT)
        ),
    )(_sc_edge_kernel)
    return call(nf, w_edge, edge_src, edge_dst, zeros)


BN = 400

def _output_body(agg_ref, ni_ref, attr_ref, deg_ref, wm_ref, wo_ref, out_ref):
    a = (agg_ref[0] + agg_ref[1]) / jnp.sqrt(deg_ref[:])
    accm = jnp.zeros((BN, D_OUT), jnp.float32)
    acco = jnp.zeros((BN, D_OUT), jnp.float32)
    for b in range(D_ATTR):
        ab = attr_ref[:, b:b + 1]
        accm += jnp.dot(ni_ref[:] * ab, wm_ref[b],
                        preferred_element_type=jnp.float32)
        acco += jnp.dot(a * ab, wo_ref[b],
                        preferred_element_type=jnp.float32)
    out_ref[:] = (C_S * NORM_IN) * accm + (C_X * NORM_IN) * acco


def _output(agg2, node_input, node_attr, node_deg, W_mask_t, W_out_t):
    return pl.pallas_call(
        _output_body,
        grid=(N // BN,),
        in_specs=[
            pl.BlockSpec((2, BN, D_IN), lambda i: (0, i, 0)),
            pl.BlockSpec((BN, D_IN), lambda i: (i, 0)),
            pl.BlockSpec((BN, D_ATTR), lambda i: (i, 0)),
            pl.BlockSpec((BN, 1), lambda i: (i, 0)),
            pl.BlockSpec((D_ATTR, D_IN, D_OUT), lambda i: (0, 0, 0)),
            pl.BlockSpec((D_ATTR, D_IN, D_OUT), lambda i: (0, 0, 0)),
        ],
        out_specs=pl.BlockSpec((BN, D_OUT), lambda i: (i, 0)),
        out_shape=jax.ShapeDtypeStruct((N, D_OUT), jnp.float32),
    )(agg2, node_input, node_attr, node_deg, W_mask_t, W_out_t)


def kernel(node_input, node_attr, node_deg, edge_src, edge_dst, edge_attr,
           edge_length_embedded, numb, n, W_in, W_mask, W1, W2, W_out):
    edge_src = edge_src.astype(jnp.int32)
    edge_dst = edge_dst.astype(jnp.int32)
    W_in_t = jnp.transpose(W_in, (1, 0, 2))
    W_mask_t = jnp.transpose(W_mask, (1, 0, 2))
    W_out_t = jnp.transpose(W_out, (1, 0, 2))

    nf, w_edge = _front(node_input, node_attr, node_deg, W_in_t,
                        edge_length_embedded, edge_attr, W1, W2)
    agg2 = _sc_aggregate(nf, w_edge, edge_src, edge_dst)
    return _output(agg2, node_input, node_attr, node_deg, W_mask_t, W_out_t)


# speedup vs baseline: 3.7020x; 1.1744x over previous
import functools
import math

import jax
import jax.numpy as jnp
from jax import lax
from jax.experimental import pallas as pl
from jax.experimental.pallas import tpu as pltpu
from jax.experimental.pallas import tpu_sc as plsc

N = 10000
E = 320000
D_IN = 128
D_ATTR = 4
D_OUT = 128
N_BASIS = 10
RADIAL = 100

NORM_IN = 1.0 / math.sqrt(D_IN * D_ATTR)
C_S = math.sin(math.pi / 8.0)
C_X = math.cos(math.pi / 8.0)

GSTEPS = 125
BN2 = N // GSTEPS
BE2 = E // GSTEPS


def _mlp_rows(elem, ea, w1_ref, w2_ref):
    h = jax.nn.silu(jnp.dot(elem, w1_ref[:],
                            preferred_element_type=jnp.float32)
                    * (1.0 / math.sqrt(N_BASIS)))
    ew = jnp.dot(h, w2_ref[:], preferred_element_type=jnp.float32)
    w_all = ew * (1.0 / math.sqrt(RADIAL)) * ea
    w16 = lax.bitcast_convert_type(w_all.astype(jnp.bfloat16), jnp.uint16)
    return w16.astype(jnp.uint32)


def _front_body(ni_ref, attr_ref, deg_ref, win_ref, elem_lo_ref, elem_hi_ref,
                ea_lo_ref, ea_hi_ref, w1_ref, w2_ref, nf_ref, w_ref):
    acc = jnp.zeros((BN2, D_IN), jnp.float32)
    for b in range(D_ATTR):
        acc += jnp.dot(ni_ref[:] * attr_ref[:, b:b + 1], win_ref[b],
                       preferred_element_type=jnp.float32)
    nf_ref[:] = acc * NORM_IN / jnp.sqrt(deg_ref[:])
    lo = _mlp_rows(elem_lo_ref[:], ea_lo_ref[:], w1_ref, w2_ref)
    hi = _mlp_rows(elem_hi_ref[:], ea_hi_ref[:], w1_ref, w2_ref)
    w_ref[:] = lax.bitcast_convert_type(lo | (hi << 16), jnp.int32)


def _front(node_input, node_attr, node_deg, W_in_t,
           edge_length_embedded, edge_attr, W1, W2):
    bp = BE2 // 2
    return pl.pallas_call(
        _front_body,
        grid=(GSTEPS,),
        in_specs=[
            pl.BlockSpec((BN2, D_IN), lambda i: (i, 0)),
            pl.BlockSpec((BN2, D_ATTR), lambda i: (i, 0)),
            pl.BlockSpec((BN2, 1), lambda i: (i, 0)),
            pl.BlockSpec((D_ATTR, D_IN, D_IN), lambda i: (0, 0, 0)),
            pl.BlockSpec((bp, N_BASIS), lambda i: (i, 0)),
            pl.BlockSpec((bp, N_BASIS), lambda i: (i + GSTEPS, 0)),
            pl.BlockSpec((bp, 1), lambda i: (i, 0)),
            pl.BlockSpec((bp, 1), lambda i: (i + GSTEPS, 0)),
            pl.BlockSpec((N_BASIS, RADIAL), lambda i: (0, 0)),
            pl.BlockSpec((RADIAL, D_IN), lambda i: (0, 0)),
        ],
        out_specs=[
            pl.BlockSpec((BN2, D_IN), lambda i: (i, 0)),
            pl.BlockSpec((bp, D_IN), lambda i: (i, 0)),
        ],
        out_shape=[
            jax.ShapeDtypeStruct((N, D_IN), jnp.float32),
            jax.ShapeDtypeStruct((E // 2, D_IN), jnp.int32),
        ],
    )(node_input, node_attr, node_deg, W_in_t,
      edge_length_embedded, edge_length_embedded, edge_attr, edge_attr,
      W1, W2)


NUM_CORES = 2
NUM_SUBCORES = 16
NUM_TILES = NUM_CORES * NUM_SUBCORES
EDGES_PER_TILE = E // NUM_TILES
CHUNK = 80
WROWS = CHUNK // 2
HALF = CHUNK // 2
PB = BE2 // 2
PROWS_PER_TILE = (E // 2) // NUM_TILES
NCHUNK = PROWS_PER_TILE // WROWS
NSLOT = 2
NIDX = 4
ROWS_PER_TILE = N // NUM_SUBCORES


def _sc_edge_kernel(nf_hbm, w_hbm, src_hbm, dst_hbm, zeros_hbm, out_hbm,
                    src_v, dst_v, g_v, w_v, agg_sh, *sems):
    sem_idx = sems[0:NIDX]
    sem_in = sems[NIDX:NIDX + NSLOT]
    sem_out = sems[NIDX + NSLOT:NIDX + NSLOT + 2]
    c = lax.axis_index("c")
    s = lax.axis_index("s")
    wid = c * NUM_SUBCORES + s
    base_p = wid * PROWS_PER_TILE

    pltpu.sync_copy(zeros_hbm, agg_sh.at[pl.ds(s * ROWS_PER_TILE, ROWS_PER_TILE)])
    plsc.subcore_barrier()

    def when(cond, fn):
        if isinstance(cond, bool):
            if cond:
                fn()
        else:
            pl.when(cond)(fn)

    def edge_offsets(i):
        lo = base_p + i * WROWS
        return lo, lo + (E // 2)

    def issue_idx(i, b):
        lo, hi = edge_offsets(i)
        pltpu.async_copy(src_hbm.at[pl.ds(lo, HALF)],
                         src_v.at[2 * b], sem_idx[b])
        pltpu.async_copy(src_hbm.at[pl.ds(hi, HALF)],
                         src_v.at[2 * b + 1], sem_idx[b])
        pltpu.async_copy(dst_hbm.at[pl.ds(lo, HALF)],
                         dst_v.at[2 * b], sem_idx[b])
        pltpu.async_copy(dst_hbm.at[pl.ds(hi, HALF)],
                         dst_v.at[2 * b + 1], sem_idx[b])

    def wait_idx(b):
        for _ in range(4):
            pltpu.make_async_copy(src_hbm.at[pl.ds(0, HALF)],
                                  src_v.at[2 * b], sem_idx[b]).wait()

    def issue_in(i, b, bi):
        pltpu.async_copy(nf_hbm.at[src_v.at[2 * bi]],
                         g_v.at[b, pl.ds(0, HALF)], sem_in[b])
        pltpu.async_copy(nf_hbm.at[src_v.at[2 * bi + 1]],
                         g_v.at[b, pl.ds(HALF, HALF)], sem_in[b])
        pltpu.async_copy(w_hbm.at[pl.ds(base_p + i * WROWS, WROWS)],
                         w_v.at[b], sem_in[b])

    def wait_in(b):
        for _ in range(2):
            pltpu.make_async_copy(nf_hbm.at[pl.ds(0, HALF)],
                                  g_v.at[b, pl.ds(0, HALF)],
                                  sem_in[b]).wait()
        pltpu.make_async_copy(w_hbm.at[pl.ds(0, WROWS)], w_v.at[b],
                              sem_in[b]).wait()

    def compute(b):
        def rowfn(r, carry2):
            for j in range(D_IN // 16):
                sl = pl.ds(j * 16, 16)
                wv = w_v[b, r, sl]
                wlo = lax.bitcast_convert_type(lax.shift_left(wv, 16),
                                               jnp.float32)
                whi = lax.bitcast_convert_type(wv & jnp.int32(-65536),
                                               jnp.float32)
                g_v[b, r, sl] = g_v[b, r, sl] * wlo
                g_v[b, HALF + r, sl] = g_v[b, HALF + r, sl] * whi
            return carry2

        lax.fori_loop(0, WROWS, rowfn, 0)

    def scatter(i_b, h, b):
        pltpu.async_copy(g_v.at[b, pl.ds(h * HALF, HALF)],
                         agg_sh.at[dst_v.at[2 * i_b + h]], sem_out[h],
                         add=True)

    def wait_out(h):
        pltpu.make_async_copy(nf_hbm.at[pl.ds(0, HALF)],
                              g_v.at[0, pl.ds(0, HALF)], sem_out[h]).wait()

    def step(i, u2, u4):
        def prefetch_data():
            wait_idx((u4 + 1) % NIDX)
            when(i >= 1, lambda: wait_out(0))
            when(i >= 1, lambda: wait_out(1))
            issue_in(i + 1, 1 - u2, (u4 + 1) % NIDX)

        when(i + 1 < NCHUNK, prefetch_data)
        wait_in(u2)
        compute(u2)
        scatter(u4, 0, u2)
        scatter(u4, 1, u2)
        when(i + 2 < NCHUNK, lambda: issue_idx(i + 2, (u4 + 2) % NIDX))

    issue_idx(0, 0)
    issue_idx(1, 1)
    wait_idx(0)
    issue_in(0, 0, 0)

    n_main = (NCHUNK - 1) // NIDX

    def body(k, carry):
        for u in range(NIDX):
            step(k * NIDX + u, u % NSLOT, u)
        return carry

    lax.fori_loop(0, n_main, body, 0)
    for i in range(n_main * NIDX, NCHUNK):
        step(i, i % NSLOT, i % NIDX)
    for _ in range(2):
        wait_out(0)
        wait_out(1)

    plsc.subcore_barrier()

    @pl.when(s == 0)
    def _():
        pltpu.sync_copy(agg_sh, out_hbm.at[c])


def _sc_aggregate(nf, w_pack, edge_src, edge_dst):
    mesh = plsc.VectorSubcoreMesh(core_axis_name="c", subcore_axis_name="s",
                                  num_cores=NUM_CORES,
                                  num_subcores=NUM_SUBCORES)
    zeros = jnp.zeros((ROWS_PER_TILE, D_IN), jnp.float32)
    call = functools.partial(
        pl.kernel,
        out_type=jax.ShapeDtypeStruct((NUM_CORES, N, D_IN), jnp.float32),
        mesh=mesh,
        scratch_types=(
            [
                pltpu.VMEM((2 * NIDX, HALF), jnp.int32),
                pltpu.VMEM((2 * NIDX, HALF), jnp.int32),
                pltpu.VMEM((NSLOT, CHUNK, D_IN), jnp.float32),
                pltpu.VMEM((NSLOT, WROWS, D_IN), jnp.int32),
                pltpu.VMEM_SHARED((N, D_IN), jnp.float32),
            ]
            + [pltpu.SemaphoreType.DMA] * (NIDX + NSLOT + 2)
        ),
    )(_sc_edge_kernel)
    return call(nf, w_pack, edge_src, edge_dst, zeros)


BN = 400

def _output_body(agg_ref, ni_ref, attr_ref, deg_ref, wm_ref, wo_ref, out_ref):
    a = (agg_ref[0] + agg_ref[1]) / jnp.sqrt(deg_ref[:])
    accm = jnp.zeros((BN, D_OUT), jnp.float32)
    acco = jnp.zeros((BN, D_OUT), jnp.float32)
    for b in range(D_ATTR):
        ab = attr_ref[:, b:b + 1]
        accm += jnp.dot(ni_ref[:] * ab, wm_ref[b],
                        preferred_element_type=jnp.float32)
        acco += jnp.dot(a * ab, wo_ref[b],
                        preferred_element_type=jnp.float32)
    out_ref[:] = (C_S * NORM_IN) * accm + (C_X * NORM_IN) * acco


def _output(agg2, node_input, node_attr, node_deg, W_mask_t, W_out_t):
    return pl.pallas_call(
        _output_body,
        grid=(N // BN,),
        in_specs=[
            pl.BlockSpec((2, BN, D_IN), lambda i: (0, i, 0)),
            pl.BlockSpec((BN, D_IN), lambda i: (i, 0)),
            pl.BlockSpec((BN, D_ATTR), lambda i: (i, 0)),
            pl.BlockSpec((BN, 1), lambda i: (i, 0)),
            pl.BlockSpec((D_ATTR, D_IN, D_OUT), lambda i: (0, 0, 0)),
            pl.BlockSpec((D_ATTR, D_IN, D_OUT), lambda i: (0, 0, 0)),
        ],
        out_specs=pl.BlockSpec((BN, D_OUT), lambda i: (i, 0)),
        out_shape=jax.ShapeDtypeStruct((N, D_OUT), jnp.float32),
    )(agg2, node_input, node_attr, node_deg, W_mask_t, W_out_t)


def kernel(node_input, node_attr, node_deg, edge_src, edge_dst, edge_attr,
           edge_length_embedded, numb, n, W_in, W_mask, W1, W2, W_out):
    edge_src = edge_src.astype(jnp.int32)
    edge_dst = edge_dst.astype(jnp.int32)
    W_in_t = jnp.transpose(W_in, (1, 0, 2))
    W_mask_t = jnp.transpose(W_mask, (1, 0, 2))
    W_out_t = jnp.transpose(W_out, (1, 0, 2))

    nf, w_edge = _front(node_input, node_attr, node_deg, W_in_t,
                        edge_length_embedded, edge_attr, W1, W2)
    agg2 = _sc_aggregate(nf, w_edge, edge_src, edge_dst)
    return _output(agg2, node_input, node_attr, node_deg, W_mask_t, W_out_t)


# speedup vs baseline: 5.3125x; 1.4350x over previous
import functools
import math

import jax
import jax.numpy as jnp
from jax import lax
from jax.experimental import pallas as pl
from jax.experimental.pallas import tpu as pltpu
from jax.experimental.pallas import tpu_sc as plsc

N = 10000
E = 320000
D_IN = 128
D_ATTR = 4
D_OUT = 128
N_BASIS = 10
RADIAL = 100

NORM_IN = 1.0 / math.sqrt(D_IN * D_ATTR)
C_S = math.sin(math.pi / 8.0)
C_X = math.cos(math.pi / 8.0)

GSTEPS = 125
BN2 = N // GSTEPS
BE2 = E // GSTEPS


def _mlp_rows(elem_t, ea_row, w1_ref, w2_ref):
    pre = lax.dot_general(elem_t, w1_ref[:], (((0,), (0,)), ((), ())),
                          preferred_element_type=jnp.float32)
    h = jax.nn.silu(pre * (1.0 / math.sqrt(N_BASIS)))
    ew = jnp.dot(h, w2_ref[:], preferred_element_type=jnp.float32)
    ea_col = lax.transpose(ea_row, (1, 0))
    w_all = ew * (1.0 / math.sqrt(RADIAL)) * ea_col
    w16 = lax.bitcast_convert_type(w_all.astype(jnp.bfloat16), jnp.uint16)
    return w16.astype(jnp.uint32)


def _front_body(ni_ref, attr_ref, deg_ref, win_ref, elem_lo_ref, elem_hi_ref,
                ea_lo_ref, ea_hi_ref, w1_ref, w2_ref, nf_ref, w_ref):
    acc = jnp.zeros((BN2, D_IN), jnp.float32)
    for b in range(D_ATTR):
        acc += jnp.dot(ni_ref[:] * attr_ref[:, b:b + 1], win_ref[b],
                       preferred_element_type=jnp.float32)
    nf_ref[:] = acc * NORM_IN / jnp.sqrt(deg_ref[:])
    lo = _mlp_rows(elem_lo_ref[:], ea_lo_ref[0], w1_ref, w2_ref)
    hi = _mlp_rows(elem_hi_ref[:], ea_hi_ref[0], w1_ref, w2_ref)
    w_ref[:] = lax.bitcast_convert_type(lo | (hi << 16), jnp.int32)


def _front(node_input, node_attr, node_deg, W_in_t,
           edge_length_embedded, edge_attr, W1, W2):
    bp = BE2 // 2
    return pl.pallas_call(
        _front_body,
        grid=(GSTEPS,),
        in_specs=[
            pl.BlockSpec((BN2, D_IN), lambda i: (i, 0)),
            pl.BlockSpec((BN2, D_ATTR), lambda i: (i, 0)),
            pl.BlockSpec((BN2, 1), lambda i: (i, 0)),
            pl.BlockSpec((D_ATTR, D_IN, D_IN), lambda i: (0, 0, 0)),
            pl.BlockSpec((N_BASIS, bp), lambda i: (0, i)),
            pl.BlockSpec((N_BASIS, bp), lambda i: (0, i + GSTEPS)),
            pl.BlockSpec((1, 1, bp), lambda i: (i, 0, 0)),
            pl.BlockSpec((1, 1, bp), lambda i: (i + GSTEPS, 0, 0)),
            pl.BlockSpec((N_BASIS, RADIAL), lambda i: (0, 0)),
            pl.BlockSpec((RADIAL, D_IN), lambda i: (0, 0)),
        ],
        out_specs=[
            pl.BlockSpec((BN2, D_IN), lambda i: (i, 0)),
            pl.BlockSpec((bp, D_IN), lambda i: (i, 0)),
        ],
        out_shape=[
            jax.ShapeDtypeStruct((N, D_IN), jnp.float32),
            jax.ShapeDtypeStruct((E // 2, D_IN), jnp.int32),
        ],
    )(node_input, node_attr, node_deg, W_in_t,
      edge_length_embedded, edge_length_embedded, edge_attr, edge_attr,
      W1, W2)


def _front_wrap(node_input, node_attr, node_deg, W_in_t,
                edge_length_embedded, edge_attr, W1, W2):
    elem_t = jnp.transpose(edge_length_embedded, (1, 0))
    ea3 = edge_attr.reshape(2 * GSTEPS, 1, BE2 // 2)
    return _front(node_input, node_attr, node_deg, W_in_t, elem_t, ea3,
                  W1, W2)


NUM_CORES = 2
NUM_SUBCORES = 16
NUM_TILES = NUM_CORES * NUM_SUBCORES
EDGES_PER_TILE = E // NUM_TILES
CHUNK = 80
WROWS = CHUNK // 2
HALF = CHUNK // 2
PB = BE2 // 2
PROWS_PER_TILE = (E // 2) // NUM_TILES
NCHUNK = PROWS_PER_TILE // WROWS
NSLOT = 2
NIDX = 4
ROWS_PER_TILE = N // NUM_SUBCORES


def _sc_edge_kernel(nf_hbm, w_hbm, src_hbm, dst_hbm, zeros_hbm, out_hbm,
                    src_v, dst_v, g_v, w_v, agg_sh, *sems):
    sem_idx = sems[0:NIDX]
    sem_in = sems[NIDX:NIDX + NSLOT]
    sem_out = sems[NIDX + NSLOT:NIDX + NSLOT + 2]
    c = lax.axis_index("c")
    s = lax.axis_index("s")
    wid = c * NUM_SUBCORES + s
    base_p = wid * PROWS_PER_TILE

    pltpu.sync_copy(zeros_hbm, agg_sh.at[pl.ds(s * ROWS_PER_TILE, ROWS_PER_TILE)])
    plsc.subcore_barrier()

    def when(cond, fn):
        if isinstance(cond, bool):
            if cond:
                fn()
        else:
            pl.when(cond)(fn)

    def edge_offsets(i):
        lo = base_p + i * WROWS
        return lo, lo + (E // 2)

    def issue_idx(i, b):
        lo, hi = edge_offsets(i)
        pltpu.async_copy(src_hbm.at[pl.ds(lo, HALF)],
                         src_v.at[2 * b], sem_idx[b])
        pltpu.async_copy(src_hbm.at[pl.ds(hi, HALF)],
                         src_v.at[2 * b + 1], sem_idx[b])
        pltpu.async_copy(dst_hbm.at[pl.ds(lo, HALF)],
                         dst_v.at[2 * b], sem_idx[b])
        pltpu.async_copy(dst_hbm.at[pl.ds(hi, HALF)],
                         dst_v.at[2 * b + 1], sem_idx[b])

    def wait_idx(b):
        for _ in range(4):
            pltpu.make_async_copy(src_hbm.at[pl.ds(0, HALF)],
                                  src_v.at[2 * b], sem_idx[b]).wait()

    def issue_in(i, b, bi):
        pltpu.async_copy(nf_hbm.at[src_v.at[2 * bi]],
                         g_v.at[b, pl.ds(0, HALF)], sem_in[b])
        pltpu.async_copy(nf_hbm.at[src_v.at[2 * bi + 1]],
                         g_v.at[b, pl.ds(HALF, HALF)], sem_in[b])
        pltpu.async_copy(w_hbm.at[pl.ds(base_p + i * WROWS, WROWS)],
                         w_v.at[b], sem_in[b])

    def wait_in(b):
        for _ in range(2):
            pltpu.make_async_copy(nf_hbm.at[pl.ds(0, HALF)],
                                  g_v.at[b, pl.ds(0, HALF)],
                                  sem_in[b]).wait()
        pltpu.make_async_copy(w_hbm.at[pl.ds(0, WROWS)], w_v.at[b],
                              sem_in[b]).wait()

    def compute(b):
        def rowfn(r, carry2):
            for j in range(D_IN // 16):
                sl = pl.ds(j * 16, 16)
                wv = w_v[b, r, sl]
                wlo = lax.bitcast_convert_type(lax.shift_left(wv, 16),
                                               jnp.float32)
                whi = lax.bitcast_convert_type(wv & jnp.int32(-65536),
                                               jnp.float32)
                g_v[b, r, sl] = g_v[b, r, sl] * wlo
                g_v[b, HALF + r, sl] = g_v[b, HALF + r, sl] * whi
            return carry2

        lax.fori_loop(0, WROWS, rowfn, 0)

    def scatter(i_b, h, b):
        pltpu.async_copy(g_v.at[b, pl.ds(h * HALF, HALF)],
                         agg_sh.at[dst_v.at[2 * i_b + h]], sem_out[h],
                         add=True)

    def wait_out(h):
        pltpu.make_async_copy(nf_hbm.at[pl.ds(0, HALF)],
                              g_v.at[0, pl.ds(0, HALF)], sem_out[h]).wait()

    def step(i, u2, u4):
        def prefetch_data():
            wait_idx((u4 + 1) % NIDX)
            when(i >= 1, lambda: wait_out(0))
            when(i >= 1, lambda: wait_out(1))
            issue_in(i + 1, 1 - u2, (u4 + 1) % NIDX)

        when(i + 1 < NCHUNK, prefetch_data)
        wait_in(u2)
        compute(u2)
        scatter(u4, 0, u2)
        scatter(u4, 1, u2)
        when(i + 2 < NCHUNK, lambda: issue_idx(i + 2, (u4 + 2) % NIDX))

    issue_idx(0, 0)
    issue_idx(1, 1)
    wait_idx(0)
    issue_in(0, 0, 0)

    n_main = (NCHUNK - 1) // NIDX

    def body(k, carry):
        for u in range(NIDX):
            step(k * NIDX + u, u % NSLOT, u)
        return carry

    lax.fori_loop(0, n_main, body, 0)
    for i in range(n_main * NIDX, NCHUNK):
        step(i, i % NSLOT, i % NIDX)
    for _ in range(2):
        wait_out(0)
        wait_out(1)

    plsc.subcore_barrier()

    @pl.when(s == 0)
    def _():
        pltpu.sync_copy(agg_sh, out_hbm.at[c])


def _sc_aggregate(nf, w_pack, edge_src, edge_dst):
    mesh = plsc.VectorSubcoreMesh(core_axis_name="c", subcore_axis_name="s",
                                  num_cores=NUM_CORES,
                                  num_subcores=NUM_SUBCORES)
    zeros = jnp.zeros((ROWS_PER_TILE, D_IN), jnp.float32)
    call = functools.partial(
        pl.kernel,
        out_type=jax.ShapeDtypeStruct((NUM_CORES, N, D_IN), jnp.float32),
        mesh=mesh,
        scratch_types=(
            [
                pltpu.VMEM((2 * NIDX, HALF), jnp.int32),
                pltpu.VMEM((2 * NIDX, HALF), jnp.int32),
                pltpu.VMEM((NSLOT, CHUNK, D_IN), jnp.float32),
                pltpu.VMEM((NSLOT, WROWS, D_IN), jnp.int32),
                pltpu.VMEM_SHARED((N, D_IN), jnp.float32),
            ]
            + [pltpu.SemaphoreType.DMA] * (NIDX + NSLOT + 2)
        ),
    )(_sc_edge_kernel)
    return call(nf, w_pack, edge_src, edge_dst, zeros)


BN = 400

def _output_body(agg_ref, ni_ref, attr_ref, deg_ref, wm_ref, wo_ref, out_ref):
    a = (agg_ref[0] + agg_ref[1]) / jnp.sqrt(deg_ref[:])
    accm = jnp.zeros((BN, D_OUT), jnp.float32)
    acco = jnp.zeros((BN, D_OUT), jnp.float32)
    for b in range(D_ATTR):
        ab = attr_ref[:, b:b + 1]
        accm += jnp.dot(ni_ref[:] * ab, wm_ref[b],
                        preferred_element_type=jnp.float32)
        acco += jnp.dot(a * ab, wo_ref[b],
                        preferred_element_type=jnp.float32)
    out_ref[:] = (C_S * NORM_IN) * accm + (C_X * NORM_IN) * acco


def _output(agg2, node_input, node_attr, node_deg, W_mask_t, W_out_t):
    return pl.pallas_call(
        _output_body,
        grid=(N // BN,),
        in_specs=[
            pl.BlockSpec((2, BN, D_IN), lambda i: (0, i, 0)),
            pl.BlockSpec((BN, D_IN), lambda i: (i, 0)),
            pl.BlockSpec((BN, D_ATTR), lambda i: (i, 0)),
            pl.BlockSpec((BN, 1), lambda i: (i, 0)),
            pl.BlockSpec((D_ATTR, D_IN, D_OUT), lambda i: (0, 0, 0)),
            pl.BlockSpec((D_ATTR, D_IN, D_OUT), lambda i: (0, 0, 0)),
        ],
        out_specs=pl.BlockSpec((BN, D_OUT), lambda i: (i, 0)),
        out_shape=jax.ShapeDtypeStruct((N, D_OUT), jnp.float32),
    )(agg2, node_input, node_attr, node_deg, W_mask_t, W_out_t)


def kernel(node_input, node_attr, node_deg, edge_src, edge_dst, edge_attr,
           edge_length_embedded, numb, n, W_in, W_mask, W1, W2, W_out):
    edge_src = edge_src.astype(jnp.int32)
    edge_dst = edge_dst.astype(jnp.int32)
    W_in_t = jnp.transpose(W_in, (1, 0, 2))
    W_mask_t = jnp.transpose(W_mask, (1, 0, 2))
    W_out_t = jnp.transpose(W_out, (1, 0, 2))

    nf, w_edge = _front_wrap(node_input, node_attr, node_deg, W_in_t,
                             edge_length_embedded, edge_attr, W1, W2)
    agg2 = _sc_aggregate(nf, w_edge, edge_src, edge_dst)
    return _output(agg2, node_input, node_attr, node_deg, W_mask_t, W_out_t)


# speedup vs baseline: 5.3790x; 1.0125x over previous
import functools
import math

import jax
import jax.numpy as jnp
from jax import lax
from jax.experimental import pallas as pl
from jax.experimental.pallas import tpu as pltpu
from jax.experimental.pallas import tpu_sc as plsc

N = 10000
E = 320000
D_IN = 128
D_ATTR = 4
D_OUT = 128
N_BASIS = 10
RADIAL = 100

NORM_IN = 1.0 / math.sqrt(D_IN * D_ATTR)
C_S = math.sin(math.pi / 8.0)
C_X = math.cos(math.pi / 8.0)

GSTEPS = 50
BN2 = N // GSTEPS
BE2 = E // GSTEPS


def _mlp_rows(elem_t, ea_row, w1_ref, w2_ref):
    pre = lax.dot_general(elem_t, w1_ref[:], (((0,), (0,)), ((), ())),
                          preferred_element_type=jnp.float32)
    h = jax.nn.silu(pre * (1.0 / math.sqrt(N_BASIS)))
    ew = jnp.dot(h, w2_ref[:], preferred_element_type=jnp.float32)
    ea_col = lax.transpose(ea_row, (1, 0))
    w_all = ew * (1.0 / math.sqrt(RADIAL)) * ea_col
    u = lax.bitcast_convert_type(w_all, jnp.uint32)
    return (u + 0x7FFF + ((u >> 16) & 1)) >> 16


def _front_body(ni_ref, attr_ref, deg_ref, win_ref, elem_lo_ref, elem_hi_ref,
                ea_lo_ref, ea_hi_ref, w1_ref, w2_ref, nf_ref, w_ref):
    acc = jnp.zeros((BN2, D_IN), jnp.float32)
    for b in range(D_ATTR):
        acc += jnp.dot(ni_ref[:] * attr_ref[:, b:b + 1], win_ref[b],
                       preferred_element_type=jnp.float32)
    nf_ref[:] = acc * NORM_IN / jnp.sqrt(deg_ref[:])
    lo = _mlp_rows(elem_lo_ref[:], ea_lo_ref[0], w1_ref, w2_ref)
    hi = _mlp_rows(elem_hi_ref[:], ea_hi_ref[0], w1_ref, w2_ref)
    w_ref[:] = lax.bitcast_convert_type(lo | (hi << 16), jnp.int32)


def _front(node_input, node_attr, node_deg, W_in_t,
           edge_length_embedded, edge_attr, W1, W2):
    bp = BE2 // 2
    return pl.pallas_call(
        _front_body,
        grid=(GSTEPS,),
        in_specs=[
            pl.BlockSpec((BN2, D_IN), lambda i: (i, 0)),
            pl.BlockSpec((BN2, D_ATTR), lambda i: (i, 0)),
            pl.BlockSpec((BN2, 1), lambda i: (i, 0)),
            pl.BlockSpec((D_ATTR, D_IN, D_IN), lambda i: (0, 0, 0)),
            pl.BlockSpec((N_BASIS, bp), lambda i: (0, i)),
            pl.BlockSpec((N_BASIS, bp), lambda i: (0, i + GSTEPS)),
            pl.BlockSpec((1, 1, bp), lambda i: (i, 0, 0)),
            pl.BlockSpec((1, 1, bp), lambda i: (i + GSTEPS, 0, 0)),
            pl.BlockSpec((N_BASIS, RADIAL), lambda i: (0, 0)),
            pl.BlockSpec((RADIAL, D_IN), lambda i: (0, 0)),
        ],
        out_specs=[
            pl.BlockSpec((BN2, D_IN), lambda i: (i, 0)),
            pl.BlockSpec((bp, D_IN), lambda i: (i, 0)),
        ],
        out_shape=[
            jax.ShapeDtypeStruct((N, D_IN), jnp.float32),
            jax.ShapeDtypeStruct((E // 2, D_IN), jnp.int32),
        ],
    )(node_input, node_attr, node_deg, W_in_t,
      edge_length_embedded, edge_length_embedded, edge_attr, edge_attr,
      W1, W2)


def _front_wrap(node_input, node_attr, node_deg, W_in_t,
                edge_length_embedded, edge_attr, W1, W2):
    elem_t = jnp.transpose(edge_length_embedded, (1, 0))
    ea3 = edge_attr.reshape(2 * GSTEPS, 1, BE2 // 2)
    return _front(node_input, node_attr, node_deg, W_in_t, elem_t, ea3,
                  W1, W2)


NUM_CORES = 2
NUM_SUBCORES = 16
NUM_TILES = NUM_CORES * NUM_SUBCORES
EDGES_PER_TILE = E // NUM_TILES
CHUNK = 80
WROWS = CHUNK // 2
HALF = CHUNK // 2
PB = BE2 // 2
PROWS_PER_TILE = (E // 2) // NUM_TILES
NCHUNK = PROWS_PER_TILE // WROWS
NSLOT = 2
NIDX = 4
ROWS_PER_TILE = N // NUM_SUBCORES


def _sc_edge_kernel(nf_hbm, w_hbm, src_hbm, dst_hbm, zeros_hbm, out_hbm,
                    src_v, dst_v, g_v, w_v, agg_sh, *sems):
    sem_idx = sems[0:NIDX]
    sem_in = sems[NIDX:NIDX + NSLOT]
    sem_out = sems[NIDX + NSLOT:NIDX + NSLOT + 2]
    c = lax.axis_index("c")
    s = lax.axis_index("s")
    wid = c * NUM_SUBCORES + s
    base_p = wid * PROWS_PER_TILE

    pltpu.sync_copy(zeros_hbm, agg_sh.at[pl.ds(s * ROWS_PER_TILE, ROWS_PER_TILE)])
    plsc.subcore_barrier()

    def when(cond, fn):
        if isinstance(cond, bool):
            if cond:
                fn()
        else:
            pl.when(cond)(fn)

    def edge_offsets(i):
        lo = base_p + i * WROWS
        return lo, lo + (E // 2)

    def issue_idx(i, b):
        lo, hi = edge_offsets(i)
        pltpu.async_copy(src_hbm.at[pl.ds(lo, HALF)],
                         src_v.at[2 * b], sem_idx[b])
        pltpu.async_copy(src_hbm.at[pl.ds(hi, HALF)],
                         src_v.at[2 * b + 1], sem_idx[b])
        pltpu.async_copy(dst_hbm.at[pl.ds(lo, HALF)],
                         dst_v.at[2 * b], sem_idx[b])
        pltpu.async_copy(dst_hbm.at[pl.ds(hi, HALF)],
                         dst_v.at[2 * b + 1], sem_idx[b])

    def wait_idx(b):
        for _ in range(4):
            pltpu.make_async_copy(src_hbm.at[pl.ds(0, HALF)],
                                  src_v.at[2 * b], sem_idx[b]).wait()

    def issue_in(i, b, bi):
        pltpu.async_copy(nf_hbm.at[src_v.at[2 * bi]],
                         g_v.at[b, pl.ds(0, HALF)], sem_in[b])
        pltpu.async_copy(nf_hbm.at[src_v.at[2 * bi + 1]],
                         g_v.at[b, pl.ds(HALF, HALF)], sem_in[b])
        pltpu.async_copy(w_hbm.at[pl.ds(base_p + i * WROWS, WROWS)],
                         w_v.at[b], sem_in[b])

    def wait_in(b):
        for _ in range(2):
            pltpu.make_async_copy(nf_hbm.at[pl.ds(0, HALF)],
                                  g_v.at[b, pl.ds(0, HALF)],
                                  sem_in[b]).wait()
        pltpu.make_async_copy(w_hbm.at[pl.ds(0, WROWS)], w_v.at[b],
                              sem_in[b]).wait()

    def compute(b):
        def rowfn(r, carry2):
            for j in range(D_IN // 16):
                sl = pl.ds(j * 16, 16)
                wv = w_v[b, r, sl]
                wlo = lax.bitcast_convert_type(lax.shift_left(wv, 16),
                                               jnp.float32)
                whi = lax.bitcast_convert_type(wv & jnp.int32(-65536),
                                               jnp.float32)
                g_v[b, r, sl] = g_v[b, r, sl] * wlo
                g_v[b, HALF + r, sl] = g_v[b, HALF + r, sl] * whi
            return carry2

        lax.fori_loop(0, WROWS, rowfn, 0)

    def scatter(i_b, h, b):
        pltpu.async_copy(g_v.at[b, pl.ds(h * HALF, HALF)],
                         agg_sh.at[dst_v.at[2 * i_b + h]], sem_out[h],
                         add=True)

    def wait_out(h):
        pltpu.make_async_copy(nf_hbm.at[pl.ds(0, HALF)],
                              g_v.at[0, pl.ds(0, HALF)], sem_out[h]).wait()

    def step(i, u2, u4):
        def prefetch_data():
            wait_idx((u4 + 1) % NIDX)
            when(i >= 1, lambda: wait_out(0))
            when(i >= 1, lambda: wait_out(1))
            issue_in(i + 1, 1 - u2, (u4 + 1) % NIDX)

        when(i + 1 < NCHUNK, prefetch_data)
        wait_in(u2)
        compute(u2)
        scatter(u4, 0, u2)
        scatter(u4, 1, u2)
        when(i + 2 < NCHUNK, lambda: issue_idx(i + 2, (u4 + 2) % NIDX))

    issue_idx(0, 0)
    issue_idx(1, 1)
    wait_idx(0)
    issue_in(0, 0, 0)

    n_main = (NCHUNK - 1) // NIDX

    def body(k, carry):
        for u in range(NIDX):
            step(k * NIDX + u, u % NSLOT, u)
        return carry

    lax.fori_loop(0, n_main, body, 0)
    for i in range(n_main * NIDX, NCHUNK):
        step(i, i % NSLOT, i % NIDX)
    for _ in range(2):
        wait_out(0)
        wait_out(1)

    plsc.subcore_barrier()

    @pl.when(s == 0)
    def _():
        pltpu.sync_copy(agg_sh, out_hbm.at[c])


def _sc_aggregate(nf, w_pack, edge_src, edge_dst):
    mesh = plsc.VectorSubcoreMesh(core_axis_name="c", subcore_axis_name="s",
                                  num_cores=NUM_CORES,
                                  num_subcores=NUM_SUBCORES)
    zeros = jnp.zeros((ROWS_PER_TILE, D_IN), jnp.float32)
    call = functools.partial(
        pl.kernel,
        out_type=jax.ShapeDtypeStruct((NUM_CORES, N, D_IN), jnp.float32),
        mesh=mesh,
        scratch_types=(
            [
                pltpu.VMEM((2 * NIDX, HALF), jnp.int32),
                pltpu.VMEM((2 * NIDX, HALF), jnp.int32),
                pltpu.VMEM((NSLOT, CHUNK, D_IN), jnp.float32),
                pltpu.VMEM((NSLOT, WROWS, D_IN), jnp.int32),
                pltpu.VMEM_SHARED((N, D_IN), jnp.float32),
            ]
            + [pltpu.SemaphoreType.DMA] * (NIDX + NSLOT + 2)
        ),
    )(_sc_edge_kernel)
    return call(nf, w_pack, edge_src, edge_dst, zeros)


BN = 400

def _output_body(agg_ref, ni_ref, attr_ref, deg_ref, wm_ref, wo_ref, out_ref):
    a = (agg_ref[0] + agg_ref[1]) / jnp.sqrt(deg_ref[:])
    accm = jnp.zeros((BN, D_OUT), jnp.float32)
    acco = jnp.zeros((BN, D_OUT), jnp.float32)
    for b in range(D_ATTR):
        ab = attr_ref[:, b:b + 1]
        accm += jnp.dot(ni_ref[:] * ab, wm_ref[b],
                        preferred_element_type=jnp.float32)
        acco += jnp.dot(a * ab, wo_ref[b],
                        preferred_element_type=jnp.float32)
    out_ref[:] = (C_S * NORM_IN) * accm + (C_X * NORM_IN) * acco


def _output(agg2, node_input, node_attr, node_deg, W_mask_t, W_out_t):
    return pl.pallas_call(
        _output_body,
        grid=(N // BN,),
        in_specs=[
            pl.BlockSpec((2, BN, D_IN), lambda i: (0, i, 0)),
            pl.BlockSpec((BN, D_IN), lambda i: (i, 0)),
            pl.BlockSpec((BN, D_ATTR), lambda i: (i, 0)),
            pl.BlockSpec((BN, 1), lambda i: (i, 0)),
            pl.BlockSpec((D_ATTR, D_IN, D_OUT), lambda i: (0, 0, 0)),
            pl.BlockSpec((D_ATTR, D_IN, D_OUT), lambda i: (0, 0, 0)),
        ],
        out_specs=pl.BlockSpec((BN, D_OUT), lambda i: (i, 0)),
        out_shape=jax.ShapeDtypeStruct((N, D_OUT), jnp.float32),
    )(agg2, node_input, node_attr, node_deg, W_mask_t, W_out_t)


def kernel(node_input, node_attr, node_deg, edge_src, edge_dst, edge_attr,
           edge_length_embedded, numb, n, W_in, W_mask, W1, W2, W_out):
    edge_src = edge_src.astype(jnp.int32)
    edge_dst = edge_dst.astype(jnp.int32)
    W_in_t = jnp.transpose(W_in, (1, 0, 2))
    W_mask_t = jnp.transpose(W_mask, (1, 0, 2))
    W_out_t = jnp.transpose(W_out, (1, 0, 2))

    nf, w_edge = _front_wrap(node_input, node_attr, node_deg, W_in_t,
                             edge_length_embedded, edge_attr, W1, W2)
    agg2 = _sc_aggregate(nf, w_edge, edge_src, edge_dst)
    return _output(agg2, node_input, node_attr, node_deg, W_mask_t, W_out_t)
